# baseline (speedup 1.0000x reference)
import functools

import jax
import jax.numpy as jnp
from jax import lax
from jax.experimental import pallas as pl
from jax.experimental.pallas import tpu as pltpu
from jax.experimental.pallas import tpu_sc as plsc

D = 1024
H = D // 2
V = 1024
NT = 4 * 2048
NW = 32
TPW = NT // NW
T = 8
R = 6 * T
NCHUNK = TPW // T
LANES = 16
RB = 128


def _spatial_body(c_hbm, w_hbm, out_hbm,
                  idx_v, ra, rb, rc, oa, ob, oc,
                  ga, gb, gc, soa, sob, soc):
    rows = (ra, rb, rc)
    outs = (oa, ob, oc)
    gsems = (ga, gb, gc)
    osems = (soa, sob, soc)
    wid = lax.axis_index("s") * 2 + lax.axis_index("c")
    base = wid * TPW

    pltpu.sync_copy(c_hbm.at[wid], idx_v)

    def gather_start(c, s):
        pltpu.async_copy(w_hbm.at[idx_v.at[pl.ds(c * R, R)]],
                         rows[s], gsems[s])

    def gather_wait(s):
        pltpu.make_async_copy(w_hbm.at[idx_v.at[pl.ds(0, R)]],
                              rows[s], gsems[s]).wait()

    def combine_store(c, s):
        r = rows[s]
        o = outs[s]
        mask_hi = jnp.uint32(0xFFFF0000)

        def tok_body(t, carry):
            def elem_body(e, carry2):
                sl = pl.ds(e * LANES, LANES)
                lo = None
                hi = None
                for j in range(6):
                    w = lax.bitcast_convert_type(r[t * 6 + j, sl],
                                                 jnp.uint32)
                    lo_j = lax.bitcast_convert_type(w << 16, jnp.float32)
                    hi_j = lax.bitcast_convert_type(w & mask_hi,
                                                    jnp.float32)
                    lo = lo_j if lo is None else lo + lo_j
                    hi = hi_j if hi is None else hi + hi_j
                o[t, sl] = lo
                o[t, pl.ds(H + e * LANES, LANES)] = hi
                return carry2
            return lax.fori_loop(0, H // LANES, elem_body, carry, unroll=8)

        lax.fori_loop(0, T, tok_body, 0)
        pltpu.async_copy(o, out_hbm.at[pl.ds(base + c * T, T)], osems[s])

    def out_wait(s):
        pltpu.make_async_copy(outs[s], out_hbm.at[pl.ds(base, T)],
                              osems[s]).wait()

    gather_start(0, 0)
    gather_start(1, 1)
    gather_start(2, 2)
    for c in range(3):
        gather_wait(c)
        combine_store(c, c)
        gather_start(c + 3, c)

    def tri_body(k, carry):
        c0 = k * 3
        for sl in range(3):
            gather_wait(sl)
            out_wait(sl)
            combine_store(c0 + sl, sl)
            gather_start(c0 + sl + 3, sl)
        return carry

    lax.fori_loop(1, 9, tri_body, 0)

    for c in range(27, NCHUNK):
        s = c % 3
        gather_wait(s)
        out_wait(s)
        combine_store(c, s)
        if c + 3 < NCHUNK:
            gather_start(c + 3, s)
    for s in range(3):
        out_wait(s)


_spatial = functools.partial(
    pl.kernel,
    mesh=plsc.VectorSubcoreMesh(core_axis_name="c", subcore_axis_name="s"),
    out_type=jax.ShapeDtypeStruct((NT, D), jnp.float32),
    scratch_types=[pltpu.VMEM((NCHUNK * R,), jnp.int32)]
                  + [pltpu.VMEM((R, H), jnp.int32) for _ in range(3)]
                  + [pltpu.VMEM((T, D), jnp.float32) for _ in range(3)]
                  + [pltpu.SemaphoreType.DMA for _ in range(6)],
)(_spatial_body)


def _pack_body(w0, w1, w2, w3, w4, w5, out):
    for j, w in enumerate((w0, w1, w2, w3, w4, w5)):
        u = lax.bitcast_convert_type(w[...], jnp.uint32)
        r = (u + jnp.uint32(0x7FFF) + ((u >> 16) & jnp.uint32(1))) >> 16
        word = r[:, :H] | (r[:, H:] << 16)
        out[j * RB:(j + 1) * RB, :] = lax.bitcast_convert_type(
            word, jnp.int32)


def _pack6(tables):
    return pl.pallas_call(
        _pack_body,
        grid=(V // RB,),
        in_specs=[pl.BlockSpec((RB, D), lambda i: (i, 0))] * 6,
        out_specs=pl.BlockSpec((6 * RB, H), lambda i: (i, 0)),
        out_shape=jax.ShapeDtypeStruct((6 * V, H), jnp.int32),
    )(*tables)


def kernel(coordinates, W_tlx, W_tly, W_brx, W_bry, W_w, W_h):
    b, s, _ = coordinates.shape
    packed = _pack6((W_tlx, W_tly, W_brx, W_bry, W_w, W_h))
    coords = coordinates.astype(jnp.int32).reshape(NT, 6)
    srow = ((coords >> 7) * (6 * RB) + (coords & (RB - 1))
            + jnp.arange(6, dtype=jnp.int32) * RB)
    out = _spatial(srow.reshape(NW, NCHUNK * R), packed)
    return out.reshape(b, s, D)


# speedup vs baseline: 1.6177x; 1.6177x over previous
import functools

import jax
import jax.numpy as jnp
from jax import lax
from jax.experimental import pallas as pl
from jax.experimental.pallas import tpu as pltpu
from jax.experimental.pallas import tpu_sc as plsc

D = 1024
H = D // 2
NT = 4 * 2048
NW = 32
TPW = NT // NW
T = 8
NCHUNK = TPW // T
LANES = 16


def _spatial_body(c_hbm, w0, w1, w2, w3, w4, w5, out_hbm,
                  idx_v, ra0, ra1, ra2, ra3, ra4, ra5,
                  rb0, rb1, rb2, rb3, rb4, rb5,
                  rc0, rc1, rc2, rc3, rc4, rc5, oa, ob, oc,
                  ga, gb, gc, soa, sob, soc):
    tabs = (w0, w1, w2, w3, w4, w5)
    rows = ((ra0, ra1, ra2, ra3, ra4, ra5),
            (rb0, rb1, rb2, rb3, rb4, rb5),
            (rc0, rc1, rc2, rc3, rc4, rc5))
    outs = (oa, ob, oc)
    gsems = (ga, gb, gc)
    osems = (soa, sob, soc)
    wid = lax.axis_index("s") * 2 + lax.axis_index("c")
    base = wid * TPW

    for j in range(6):
        pltpu.sync_copy(c_hbm.at[j, pl.ds(base, TPW)], idx_v.at[j])

    def gather_start(c, s):
        for j in range(6):
            pltpu.async_copy(tabs[j].at[idx_v.at[j, pl.ds(c * T, T)]],
                             rows[s][j], gsems[s])

    def gather_wait(s):
        for j in range(6):
            pltpu.make_async_copy(tabs[j].at[idx_v.at[j, pl.ds(0, T)]],
                                  rows[s][j], gsems[s]).wait()

    def combine_store(c, s):
        o = outs[s]
        mask_hi = jnp.uint32(0xFFFF0000)

        def tok_body(t, carry):
            def elem_body(e, carry2):
                sl = pl.ds(e * LANES, LANES)
                lo = None
                hi = None
                for j in range(6):
                    w = lax.bitcast_convert_type(rows[s][j][t, sl],
                                                 jnp.uint32)
                    lo_j = lax.bitcast_convert_type(w << 16, jnp.float32)
                    hi_j = lax.bitcast_convert_type(w & mask_hi,
                                                    jnp.float32)
                    lo = lo_j if lo is None else lo + lo_j
                    hi = hi_j if hi is None else hi + hi_j
                o[t, sl] = lo
                o[t, pl.ds(H + e * LANES, LANES)] = hi
                return carry2
            return lax.fori_loop(0, H // LANES, elem_body, carry, unroll=8)

        lax.fori_loop(0, T, tok_body, 0)
        pltpu.async_copy(o, out_hbm.at[pl.ds(base + c * T, T)], osems[s])

    def out_wait(s):
        pltpu.make_async_copy(outs[s], out_hbm.at[pl.ds(base, T)],
                              osems[s]).wait()

    gather_start(0, 0)
    gather_start(1, 1)
    gather_start(2, 2)
    for c in range(3):
        gather_wait(c)
        combine_store(c, c)
        gather_start(c + 3, c)

    def tri_body(k, carry):
        c0 = k * 3
        for sl in range(3):
            gather_wait(sl)
            out_wait(sl)
            combine_store(c0 + sl, sl)
            gather_start(c0 + sl + 3, sl)
        return carry

    lax.fori_loop(1, 9, tri_body, 0)

    for c in range(27, NCHUNK):
        s = c % 3
        gather_wait(s)
        out_wait(s)
        combine_store(c, s)
        if c + 3 < NCHUNK:
            gather_start(c + 3, s)
    for s in range(3):
        out_wait(s)


_spatial = functools.partial(
    pl.kernel,
    mesh=plsc.VectorSubcoreMesh(core_axis_name="c", subcore_axis_name="s"),
    out_type=jax.ShapeDtypeStruct((NT, D), jnp.float32),
    scratch_types=[pltpu.VMEM((6, TPW), jnp.int32)]
                  + [pltpu.VMEM((T, H), jnp.int32) for _ in range(18)]
                  + [pltpu.VMEM((T, D), jnp.float32) for _ in range(3)]
                  + [pltpu.SemaphoreType.DMA for _ in range(6)],
)(_spatial_body)


def _pack_body(w0, w1, w2, w3, w4, w5, o0, o1, o2, o3, o4, o5):
    for w, o in ((w0, o0), (w1, o1), (w2, o2), (w3, o3), (w4, o4),
                 (w5, o5)):
        u = lax.bitcast_convert_type(w[...], jnp.uint32)
        r = (u + jnp.uint32(0x7FFF) + ((u >> 16) & jnp.uint32(1))) >> 16
        word = r[:, :H] | (r[:, H:] << 16)
        o[...] = lax.bitcast_convert_type(word, jnp.int32)


_PACK_BR = 128


def _pack6(tables):
    n = tables[0].shape[0]
    return pl.pallas_call(
        _pack_body,
        grid=(n // _PACK_BR,),
        in_specs=[pl.BlockSpec((_PACK_BR, D), lambda i: (i, 0))] * 6,
        out_specs=[pl.BlockSpec((_PACK_BR, H), lambda i: (i, 0))] * 6,
        out_shape=[jax.ShapeDtypeStruct((n, H), jnp.int32)] * 6,
    )(*tables)


def kernel(coordinates, W_tlx, W_tly, W_brx, W_bry, W_w, W_h):
    b, s, _ = coordinates.shape
    coords = coordinates.astype(jnp.int32).reshape(NT, 6).T
    packed = _pack6((W_tlx, W_tly, W_brx, W_bry, W_w, W_h))
    out = _spatial(coords, *packed)
    return out.reshape(b, s, D)
